# Initial kernel scaffold; baseline (speedup 1.0000x reference)
import jax
import jax.numpy as jnp
from jax.experimental import pallas as pl


def kernel(x, edge_index, edge_attr, ln1_g, ln1_b, W0, b0, W1, b1, n1_g, n1_b, ln2_g, ln2_b, We, be, Wf, bf):
    raise NotImplementedError("write your pallas kernel here")


# speedup vs baseline: 4.0348x; 4.0348x over previous
import functools

import jax
import jax.numpy as jnp
from jax import lax
from jax.experimental import pallas as pl
from jax.experimental.pallas import tpu as pltpu
from jax.experimental.pallas import tpu_sc as plsc

NC, NS, L = 2, 16, 16
NW = NC * NS
EB = 128
NBLK = 128


def _zero_rows(buf, nrows, ncols):
    @pl.loop(0, nrows)
    def _(i):
        for f in range(ncols // L):
            buf[i, pl.ds(f * L, L)] = jnp.zeros((L,), jnp.float32)


def _sc_deg(npad, ept, nb, col_hbm, ew_hbm, out_hbm, acc, cidx, ewb, zbuf, sem):
    c = lax.axis_index("c")
    s = lax.axis_index("s")
    wid = c * NS + s
    rpt = npad // NS
    for g in range(EB // L):
        zbuf[pl.ds(g * L, L)] = jnp.zeros((L,), jnp.float32)
    @pl.loop(0, rpt // EB)
    def _(k):
        pltpu.sync_copy(zbuf, acc.at[pl.ds(s * rpt + k * EB, EB)])
    plsc.subcore_barrier()

    @pl.loop(0, nb)
    def _(b):
        base = wid * ept + b * EB
        pltpu.sync_copy(col_hbm.at[pl.ds(base, EB)], cidx)
        pltpu.sync_copy(ew_hbm.at[pl.ds(base, EB)], ewb)
        pltpu.sync_copy(ewb, acc.at[cidx], add=True)

    plsc.subcore_barrier()
    pltpu.sync_copy(acc.at[pl.ds(s * rpt, rpt)],
                    out_hbm.at[pl.ds(c * npad + s * rpt, rpt)])


def _sc_mp(npad, ept, nb, y_hbm, row_hbm, col_hbm, ew_hbm, out_hbm,
           acc, ridx, cidx, ewb, rows, sem):
    c = lax.axis_index("c")
    s = lax.axis_index("s")
    wid = c * NS + s
    rpt = npad // NS
    _zero_rows(rows, EB, 128)
    @pl.loop(0, rpt // EB)
    def _(k):
        pltpu.sync_copy(rows, acc.at[pl.ds(s * rpt + k * EB, EB)])
    plsc.subcore_barrier()

    @pl.loop(0, nb)
    def _(b):
        base = wid * ept + b * EB
        pltpu.sync_copy(row_hbm.at[pl.ds(base, EB)], ridx)
        pltpu.sync_copy(col_hbm.at[pl.ds(base, EB)], cidx)
        pltpu.sync_copy(ew_hbm.at[pl.ds(base, EB)], ewb)
        pltpu.async_copy(y_hbm.at[ridx], rows, sem).wait()
        @pl.loop(0, EB // L)
        def _(g):
            w16 = ewb[pl.ds(g * L, L)]
            for j in range(L):
                w = w16[j]
                r = g * L + j
                for f in range(8):
                    sl = pl.ds(f * L, L)
                    rows[r, sl] = rows[r, sl] * w
        pltpu.sync_copy(rows, acc.at[cidx], add=True)

    plsc.subcore_barrier()
    pltpu.sync_copy(acc.at[pl.ds(s * rpt, rpt)],
                    out_hbm.at[pl.ds(c * npad + s * rpt, rpt)])


def _sc_readout(ept, nb, h2_hbm, row_hbm, col_hbm, hr_hbm, hc_hbm,
                ridx, cidx, bufr, bufc, sem0, sem1):
    c = lax.axis_index("c")
    s = lax.axis_index("s")
    wid = c * NS + s

    @pl.loop(0, nb)
    def _(b):
        base = wid * ept + b * EB
        pltpu.sync_copy(row_hbm.at[pl.ds(base, EB)], ridx)
        pltpu.sync_copy(col_hbm.at[pl.ds(base, EB)], cidx)
        d0 = pltpu.async_copy(h2_hbm.at[ridx], bufr, sem0)
        d1 = pltpu.async_copy(h2_hbm.at[cidx], bufc, sem1)
        d0.wait()
        d1.wait()
        pltpu.sync_copy(bufr, hr_hbm.at[pl.ds(base, EB)])
        pltpu.sync_copy(bufc, hc_hbm.at[pl.ds(base, EB)])


def _leaky(x):
    return jnp.where(x > 0, x, 0.01 * x)


def _dinv_of(dg0, dg1):
    deg = dg0[...] + dg1[...] + 1.0
    return jnp.where(deg > 0, lax.rsqrt(deg), 0.0)


def _ln(x, g, b):
    mu = jnp.mean(x, axis=-1, keepdims=True)
    var = jnp.mean((x - mu) * (x - mu), axis=-1, keepdims=True)
    return (x - mu) * lax.rsqrt(var + 1e-5) * g + b


def _tc_a(x_ref, dg0, dg1, g1, b1, w0, y0_out):
    xb = x_ref[...]
    ln = _ln(xb, g1[...], b1[...])
    xw = jnp.dot(ln, w0[...], preferred_element_type=jnp.float32)
    dinv = _dinv_of(dg0[...], dg1[...])
    y0_out[...] = dinv[:, None] * xw


def _tc_b(a0, a1, y0, dg0, dg1, b0, w1, h_out, y1_out):
    dinv = _dinv_of(dg0[...], dg1[...])
    hpre = dinv[:, None] * (a0[...] + a1[...] + y0[...]) + b0[...]
    h = _leaky(hpre)
    h_out[...] = h
    xw1 = jnp.dot(h, w1[...], preferred_element_type=jnp.float32)
    y1_out[...] = dinv[:, None] * xw1


def _tc_c(a0, a1, y1, h_ref, dg0, dg1, b1, n1g, n1b, h2_out):
    dinv = _dinv_of(dg0[...], dg1[...])
    g = dinv[:, None] * (a0[...] + a1[...] + y1[...]) + b1[...]
    h2_out[...] = _leaky(_ln(g, n1g[...], n1b[...]) + h_ref[...])


def _tc_d(hr, hc, lg2, lb2, we, be, wf, bf, out):
    ef = jnp.concatenate([hr[...], hc[...]], axis=1)
    ln = _ln(ef, lg2[...], lb2[...])
    eo1 = _leaky(jnp.dot(ln, we[...], preferred_element_type=jnp.float32)
                 + be[...])
    o = jnp.sum(eo1 * wf[...], axis=1)
    out[...] = o[:, None] + bf[0, 0]


def kernel(x, edge_index, edge_attr, ln1_g, ln1_b, W0, b0, W1, b1,
           n1_g, n1_b, ln2_g, ln2_b, We, be, Wf, bf):
    N, D = x.shape
    E = edge_attr.shape[0]
    H = W0.shape[1]
    npad = ((N + NS * EB - 1) // (NS * EB)) * (NS * EB)
    e_pad = ((E + NW * EB - 1) // (NW * EB)) * (NW * EB)
    ept = e_pad // NW
    nb = ept // EB

    f32 = jnp.float32
    rowp = jnp.pad(edge_index[0], (0, e_pad - E))
    colp = jnp.pad(edge_index[1], (0, e_pad - E))
    ewp = jnp.pad(edge_attr, (0, e_pad - E))
    xp = jnp.pad(x, ((0, npad - N), (0, 0)))

    mesh = plsc.VectorSubcoreMesh(core_axis_name="c", subcore_axis_name="s",
                                  num_cores=NC, num_subcores=NS)

    deg_parts = pl.kernel(
        functools.partial(_sc_deg, npad, ept, nb),
        out_type=jax.ShapeDtypeStruct((NC * npad,), f32),
        mesh=mesh,
        scratch_types=[
            pltpu.VMEM_SHARED((npad,), f32),
            pltpu.VMEM((EB,), jnp.int32),
            pltpu.VMEM((EB,), f32),
            pltpu.VMEM((EB,), f32),
            pltpu.SemaphoreType.DMA,
        ],
    )(colp, ewp)
    dg0 = deg_parts[:npad]
    dg1 = deg_parts[npad:]

    nblocks = npad // NBLK
    bspec_n = pl.BlockSpec((NBLK, H), lambda i: (i, 0))
    bspec_16 = pl.BlockSpec((NBLK,), lambda i: (i,))
    bspec_st = pl.BlockSpec((NBLK, 16), lambda i: (i, 0))
    bspec_p = pl.BlockSpec((1, H), lambda i: (0, 0))
    bspec_w = pl.BlockSpec((D, H), lambda i: (0, 0))

    y0 = pl.pallas_call(
        _tc_a,
        grid=(nblocks,),
        in_specs=[bspec_n, bspec_16, bspec_16, bspec_p, bspec_p, bspec_w],
        out_specs=bspec_n,
        out_shape=jax.ShapeDtypeStruct((npad, H), f32),
    )(xp, dg0, dg1, ln1_g.reshape(1, -1), ln1_b.reshape(1, -1), W0)

    def mp(y):
        parts = pl.kernel(
            functools.partial(_sc_mp, npad, ept, nb),
            out_type=jax.ShapeDtypeStruct((NC * npad, H), f32),
            mesh=mesh,
            scratch_types=[
                pltpu.VMEM_SHARED((npad, H), f32),
                pltpu.VMEM((EB,), jnp.int32),
                pltpu.VMEM((EB,), jnp.int32),
                pltpu.VMEM((EB,), f32),
                pltpu.VMEM((EB, H), f32),
                pltpu.SemaphoreType.DMA,
            ],
        )(y, rowp, colp, ewp)
        return parts[:npad], parts[npad:]

    a0, a1 = mp(y0)
    h, y1 = pl.pallas_call(
        _tc_b,
        grid=(nblocks,),
        in_specs=[bspec_n, bspec_n, bspec_n, bspec_16, bspec_16, bspec_p,
                  bspec_w],
        out_specs=[bspec_n, bspec_n],
        out_shape=[jax.ShapeDtypeStruct((npad, H), f32),
                   jax.ShapeDtypeStruct((npad, H), f32)],
    )(a0, a1, y0, dg0, dg1, b0.reshape(1, -1), W1)

    c0, c1 = mp(y1)
    h2 = pl.pallas_call(
        _tc_c,
        grid=(nblocks,),
        in_specs=[bspec_n, bspec_n, bspec_n, bspec_n, bspec_16, bspec_16,
                  bspec_p, bspec_p, bspec_p],
        out_specs=bspec_n,
        out_shape=jax.ShapeDtypeStruct((npad, H), f32),
    )(c0, c1, y1, h, dg0, dg1, b1.reshape(1, -1), n1_g.reshape(1, -1),
      n1_b.reshape(1, -1))

    hr, hc = pl.kernel(
        functools.partial(_sc_readout, ept, nb),
        out_type=[jax.ShapeDtypeStruct((e_pad, H), f32),
                  jax.ShapeDtypeStruct((e_pad, H), f32)],
        mesh=mesh,
        scratch_types=[
            pltpu.VMEM((EB,), jnp.int32),
            pltpu.VMEM((EB,), jnp.int32),
            pltpu.VMEM((EB, H), f32),
            pltpu.VMEM((EB, H), f32),
            pltpu.SemaphoreType.DMA,
            pltpu.SemaphoreType.DMA,
        ],
    )(h2, rowp, colp)

    eblk = 512
    egrid = e_pad // eblk
    espec = pl.BlockSpec((eblk, H), lambda i: (i, 0))
    cspec = pl.BlockSpec((1, 2 * H), lambda i: (0, 0))
    eo = pl.pallas_call(
        _tc_d,
        grid=(egrid,),
        in_specs=[espec, espec, cspec, cspec,
                  pl.BlockSpec((2 * H, H), lambda i: (0, 0)),
                  pl.BlockSpec((1, H), lambda i: (0, 0)),
                  pl.BlockSpec((1, H), lambda i: (0, 0)),
                  pl.BlockSpec((1, 1), lambda i: (0, 0))],
        out_specs=pl.BlockSpec((eblk, 1), lambda i: (i, 0)),
        out_shape=jax.ShapeDtypeStruct((e_pad, 1), f32),
    )(hr, hc, ln2_g.reshape(1, -1), ln2_b.reshape(1, -1), We,
      be.reshape(1, -1), Wf.reshape(1, -1), bf.reshape(1, 1))

    return eo[:E]


# speedup vs baseline: 4.8198x; 1.1946x over previous
import functools

import jax
import jax.numpy as jnp
from jax import lax
from jax.experimental import pallas as pl
from jax.experimental.pallas import tpu as pltpu
from jax.experimental.pallas import tpu_sc as plsc

NC, NS, L = 2, 16, 16
NW = NC * NS
EB = 128
NBLK = 128


def _zero_rows(buf, nrows, ncols):
    @pl.loop(0, nrows)
    def _(i):
        for f in range(ncols // L):
            buf[i, pl.ds(f * L, L)] = jnp.zeros((L,), jnp.float32)


def _sc_deg(npad, ept, nb, col_hbm, ew_hbm, out_hbm, acc, cidx, ewb, zbuf, sem):
    c = lax.axis_index("c")
    s = lax.axis_index("s")
    wid = c * NS + s
    rpt = npad // NS
    for g in range(EB // L):
        zbuf[pl.ds(g * L, L)] = jnp.zeros((L,), jnp.float32)
    @pl.loop(0, rpt // EB)
    def _(k):
        pltpu.sync_copy(zbuf, acc.at[pl.ds(s * rpt + k * EB, EB)])
    plsc.subcore_barrier()

    @pl.loop(0, nb)
    def _(b):
        base = wid * ept + b * EB
        pltpu.sync_copy(col_hbm.at[pl.ds(base, EB)], cidx)
        pltpu.sync_copy(ew_hbm.at[pl.ds(base, EB)], ewb)
        pltpu.sync_copy(ewb, acc.at[cidx], add=True)

    plsc.subcore_barrier()
    pltpu.sync_copy(acc.at[pl.ds(s * rpt, rpt)],
                    out_hbm.at[pl.ds(c * npad + s * rpt, rpt)])


def _sc_mp(npad, ept, nb, y_hbm, row_hbm, col_hbm, ew_hbm, out_hbm,
           acc, ridx, cidx, ewb, rows0, rows1, sem0, sem1):
    c = lax.axis_index("c")
    s = lax.axis_index("s")
    wid = c * NS + s
    rpt = npad // NS
    base = wid * ept
    _zero_rows(rows0, EB, 128)
    @pl.loop(0, rpt // EB)
    def _(k):
        pltpu.sync_copy(rows0, acc.at[pl.ds(s * rpt + k * EB, EB)])
    plsc.subcore_barrier()

    ridx0, ridx1 = ridx
    cidx0, cidx1 = cidx
    ewb0, ewb1 = ewb

    def scale(rows, ew):
        @pl.loop(0, EB // L)
        def _(g):
            w16 = ew[pl.ds(g * L, L)]
            for j in range(L):
                w = w16[j]
                r = g * L + j
                for f in range(8):
                    sl = pl.ds(f * L, L)
                    rows[r, sl] = rows[r, sl] * w

    def start(b, ri, ci, ew, rows, sem):
        eb = base + b * EB
        pltpu.sync_copy(row_hbm.at[pl.ds(eb, EB)], ri)
        pltpu.sync_copy(col_hbm.at[pl.ds(eb, EB)], ci)
        pltpu.sync_copy(ew_hbm.at[pl.ds(eb, EB)], ew)
        pltpu.async_copy(y_hbm.at[ri], rows, sem)

    def wait(ri, rows, sem):
        pltpu.make_async_copy(y_hbm.at[ri], rows, sem).wait()

    start(0, ridx0, cidx0, ewb0, rows0, sem0)
    @pl.loop(0, nb // 2)
    def _(k):
        b0 = 2 * k
        start(b0 + 1, ridx1, cidx1, ewb1, rows1, sem1)
        wait(ridx0, rows0, sem0)
        scale(rows0, ewb0)
        pltpu.sync_copy(rows0, acc.at[cidx0], add=True)
        @pl.when(b0 + 2 < nb)
        def _():
            start(b0 + 2, ridx0, cidx0, ewb0, rows0, sem0)
        wait(ridx1, rows1, sem1)
        scale(rows1, ewb1)
        pltpu.sync_copy(rows1, acc.at[cidx1], add=True)
    if nb % 2 == 1:
        wait(ridx0, rows0, sem0)
        scale(rows0, ewb0)
        pltpu.sync_copy(rows0, acc.at[cidx0], add=True)

    plsc.subcore_barrier()
    pltpu.sync_copy(acc.at[pl.ds(s * rpt, rpt)],
                    out_hbm.at[pl.ds(c * npad + s * rpt, rpt)])


def _sc_readout(ept, nb, h2_hbm, row_hbm, col_hbm, hr_hbm, hc_hbm,
                ridx, cidx, bufr0, bufc0, bufr1, bufc1, semr0, semc0,
                semr1, semc1):
    c = lax.axis_index("c")
    s = lax.axis_index("s")
    wid = c * NS + s
    base = wid * ept
    pltpu.sync_copy(row_hbm.at[pl.ds(base, ept)], ridx)
    pltpu.sync_copy(col_hbm.at[pl.ds(base, ept)], cidx)

    def start(b, br, bc, sr, sc_):
        pltpu.async_copy(h2_hbm.at[ridx.at[pl.ds(b * EB, EB)]], br, sr)
        pltpu.async_copy(h2_hbm.at[cidx.at[pl.ds(b * EB, EB)]], bc, sc_)

    def finish(b, br, bc, sr, sc_):
        pltpu.make_async_copy(h2_hbm.at[ridx.at[pl.ds(0, EB)]], br, sr).wait()
        pltpu.make_async_copy(h2_hbm.at[cidx.at[pl.ds(0, EB)]], bc, sc_).wait()
        pltpu.sync_copy(br, hr_hbm.at[pl.ds(base + b * EB, EB)])
        pltpu.sync_copy(bc, hc_hbm.at[pl.ds(base + b * EB, EB)])

    start(0, bufr0, bufc0, semr0, semc0)
    @pl.loop(0, nb // 2)
    def _(k):
        b0 = 2 * k
        start(b0 + 1, bufr1, bufc1, semr1, semc1)
        finish(b0, bufr0, bufc0, semr0, semc0)
        @pl.when(b0 + 2 < nb)
        def _():
            start(b0 + 2, bufr0, bufc0, semr0, semc0)
        finish(b0 + 1, bufr1, bufc1, semr1, semc1)
    if nb % 2 == 1:
        finish(nb - 1, bufr0, bufc0, semr0, semc0)


def _leaky(x):
    return jnp.where(x > 0, x, 0.01 * x)


def _dinv_of(dg0, dg1):
    deg = dg0[...] + dg1[...] + 1.0
    return jnp.where(deg > 0, lax.rsqrt(deg), 0.0)


def _ln(x, g, b):
    mu = jnp.mean(x, axis=-1, keepdims=True)
    var = jnp.mean((x - mu) * (x - mu), axis=-1, keepdims=True)
    return (x - mu) * lax.rsqrt(var + 1e-5) * g + b


def _tc_a(x_ref, dg0, dg1, g1, b1, w0, y0_out):
    xb = x_ref[...]
    ln = _ln(xb, g1[...], b1[...])
    xw = jnp.dot(ln, w0[...], preferred_element_type=jnp.float32)
    dinv = _dinv_of(dg0[...], dg1[...])
    y0_out[...] = dinv[:, None] * xw


def _tc_b(a0, a1, y0, dg0, dg1, b0, w1, h_out, y1_out):
    dinv = _dinv_of(dg0[...], dg1[...])
    hpre = dinv[:, None] * (a0[...] + a1[...] + y0[...]) + b0[...]
    h = _leaky(hpre)
    h_out[...] = h
    xw1 = jnp.dot(h, w1[...], preferred_element_type=jnp.float32)
    y1_out[...] = dinv[:, None] * xw1


def _tc_c(a0, a1, y1, h_ref, dg0, dg1, b1, n1g, n1b, h2_out):
    dinv = _dinv_of(dg0[...], dg1[...])
    g = dinv[:, None] * (a0[...] + a1[...] + y1[...]) + b1[...]
    h2_out[...] = _leaky(_ln(g, n1g[...], n1b[...]) + h_ref[...])


def _tc_d(hr, hc, lg2, lb2, we, be, wf, bf, out):
    ef = jnp.concatenate([hr[...], hc[...]], axis=1)
    ln = _ln(ef, lg2[...], lb2[...])
    eo1 = _leaky(jnp.dot(ln, we[...], preferred_element_type=jnp.float32)
                 + be[...])
    o = jnp.sum(eo1 * wf[...], axis=1)
    out[...] = o[:, None] + bf[0, 0]


def kernel(x, edge_index, edge_attr, ln1_g, ln1_b, W0, b0, W1, b1,
           n1_g, n1_b, ln2_g, ln2_b, We, be, Wf, bf):
    N, D = x.shape
    E = edge_attr.shape[0]
    H = W0.shape[1]
    npad = ((N + NS * EB - 1) // (NS * EB)) * (NS * EB)
    e_pad = ((E + NW * EB - 1) // (NW * EB)) * (NW * EB)
    ept = e_pad // NW
    nb = ept // EB

    f32 = jnp.float32
    rowp = jnp.pad(edge_index[0], (0, e_pad - E))
    colp = jnp.pad(edge_index[1], (0, e_pad - E))
    ewp = jnp.pad(edge_attr, (0, e_pad - E))
    xp = jnp.pad(x, ((0, npad - N), (0, 0)))

    mesh = plsc.VectorSubcoreMesh(core_axis_name="c", subcore_axis_name="s",
                                  num_cores=NC, num_subcores=NS)

    deg_parts = pl.kernel(
        functools.partial(_sc_deg, npad, ept, nb),
        out_type=jax.ShapeDtypeStruct((NC * npad,), f32),
        mesh=mesh,
        scratch_types=[
            pltpu.VMEM_SHARED((npad,), f32),
            pltpu.VMEM((EB,), jnp.int32),
            pltpu.VMEM((EB,), f32),
            pltpu.VMEM((EB,), f32),
            pltpu.SemaphoreType.DMA,
        ],
    )(colp, ewp)
    dg0 = deg_parts[:npad]
    dg1 = deg_parts[npad:]

    nblocks = npad // NBLK
    bspec_n = pl.BlockSpec((NBLK, H), lambda i: (i, 0))
    bspec_16 = pl.BlockSpec((NBLK,), lambda i: (i,))
    bspec_st = pl.BlockSpec((NBLK, 16), lambda i: (i, 0))
    bspec_p = pl.BlockSpec((1, H), lambda i: (0, 0))
    bspec_w = pl.BlockSpec((D, H), lambda i: (0, 0))

    y0 = pl.pallas_call(
        _tc_a,
        grid=(nblocks,),
        in_specs=[bspec_n, bspec_16, bspec_16, bspec_p, bspec_p, bspec_w],
        out_specs=bspec_n,
        out_shape=jax.ShapeDtypeStruct((npad, H), f32),
    )(xp, dg0, dg1, ln1_g.reshape(1, -1), ln1_b.reshape(1, -1), W0)

    def mp(y):
        parts = pl.kernel(
            functools.partial(_sc_mp, npad, ept, nb),
            out_type=jax.ShapeDtypeStruct((NC * npad, H), f32),
            mesh=mesh,
            scratch_types=[
                pltpu.VMEM_SHARED((npad, H), f32),
                (pltpu.VMEM((EB,), jnp.int32), pltpu.VMEM((EB,), jnp.int32)),
                (pltpu.VMEM((EB,), jnp.int32), pltpu.VMEM((EB,), jnp.int32)),
                (pltpu.VMEM((EB,), f32), pltpu.VMEM((EB,), f32)),
                pltpu.VMEM((EB, H), f32),
                pltpu.VMEM((EB, H), f32),
                pltpu.SemaphoreType.DMA,
                pltpu.SemaphoreType.DMA,
            ],
        )(y, rowp, colp, ewp)
        return parts[:npad], parts[npad:]

    a0, a1 = mp(y0)
    h, y1 = pl.pallas_call(
        _tc_b,
        grid=(nblocks,),
        in_specs=[bspec_n, bspec_n, bspec_n, bspec_16, bspec_16, bspec_p,
                  bspec_w],
        out_specs=[bspec_n, bspec_n],
        out_shape=[jax.ShapeDtypeStruct((npad, H), f32),
                   jax.ShapeDtypeStruct((npad, H), f32)],
    )(a0, a1, y0, dg0, dg1, b0.reshape(1, -1), W1)

    c0, c1 = mp(y1)
    h2 = pl.pallas_call(
        _tc_c,
        grid=(nblocks,),
        in_specs=[bspec_n, bspec_n, bspec_n, bspec_n, bspec_16, bspec_16,
                  bspec_p, bspec_p, bspec_p],
        out_specs=bspec_n,
        out_shape=jax.ShapeDtypeStruct((npad, H), f32),
    )(c0, c1, y1, h, dg0, dg1, b1.reshape(1, -1), n1_g.reshape(1, -1),
      n1_b.reshape(1, -1))

    hr, hc = pl.kernel(
        functools.partial(_sc_readout, ept, nb),
        out_type=[jax.ShapeDtypeStruct((e_pad, H), f32),
                  jax.ShapeDtypeStruct((e_pad, H), f32)],
        mesh=mesh,
        scratch_types=[
            pltpu.VMEM((ept,), jnp.int32),
            pltpu.VMEM((ept,), jnp.int32),
            pltpu.VMEM((EB, H), f32),
            pltpu.VMEM((EB, H), f32),
            pltpu.VMEM((EB, H), f32),
            pltpu.VMEM((EB, H), f32),
            pltpu.SemaphoreType.DMA,
            pltpu.SemaphoreType.DMA,
            pltpu.SemaphoreType.DMA,
            pltpu.SemaphoreType.DMA,
        ],
    )(h2, rowp, colp)

    eblk = 512
    egrid = e_pad // eblk
    espec = pl.BlockSpec((eblk, H), lambda i: (i, 0))
    cspec = pl.BlockSpec((1, 2 * H), lambda i: (0, 0))
    eo = pl.pallas_call(
        _tc_d,
        grid=(egrid,),
        in_specs=[espec, espec, cspec, cspec,
                  pl.BlockSpec((2 * H, H), lambda i: (0, 0)),
                  pl.BlockSpec((1, H), lambda i: (0, 0)),
                  pl.BlockSpec((1, H), lambda i: (0, 0)),
                  pl.BlockSpec((1, 1), lambda i: (0, 0))],
        out_specs=pl.BlockSpec((eblk, 1), lambda i: (i, 0)),
        out_shape=jax.ShapeDtypeStruct((e_pad, 1), f32),
    )(hr, hc, ln2_g.reshape(1, -1), ln2_b.reshape(1, -1), We,
      be.reshape(1, -1), Wf.reshape(1, -1), bf.reshape(1, 1))

    return eo[:E]


# speedup vs baseline: 5.2146x; 1.0819x over previous
import functools

import jax
import jax.numpy as jnp
from jax import lax
from jax.experimental import pallas as pl
from jax.experimental.pallas import tpu as pltpu
from jax.experimental.pallas import tpu_sc as plsc

NC, NS, L = 2, 16, 16
NW = NC * NS
EB = 128
NBLK = 128


def _zero_rows(buf, nrows, ncols):
    @pl.loop(0, nrows)
    def _(i):
        for f in range(ncols // L):
            buf[i, pl.ds(f * L, L)] = jnp.zeros((L,), jnp.float32)


def _sc_deg(npad, ept, nb, col_hbm, ew_hbm, out_hbm, acc, cidx, ewb, zbuf, sem):
    c = lax.axis_index("c")
    s = lax.axis_index("s")
    wid = c * NS + s
    rpt = npad // NS
    for g in range(EB // L):
        zbuf[pl.ds(g * L, L)] = jnp.zeros((L,), jnp.float32)
    @pl.loop(0, rpt // EB)
    def _(k):
        pltpu.sync_copy(zbuf, acc.at[pl.ds(s * rpt + k * EB, EB)])
    plsc.subcore_barrier()

    @pl.loop(0, nb)
    def _(b):
        base = wid * ept + b * EB
        pltpu.sync_copy(col_hbm.at[pl.ds(base, EB)], cidx)
        pltpu.sync_copy(ew_hbm.at[pl.ds(base, EB)], ewb)
        pltpu.sync_copy(ewb, acc.at[cidx], add=True)

    plsc.subcore_barrier()
    pltpu.sync_copy(acc.at[pl.ds(s * rpt, rpt)],
                    out_hbm.at[pl.ds(c * npad + s * rpt, rpt)])


def _sc_mp(npad, nb0, nb1, y_hbm, row_hbm, col_hbm, ew_hbm, out_hbm,
           acc, ridx, cidx, ewb, rows0, rows1, sem0, sem1):
    c = lax.axis_index("c")
    s = lax.axis_index("s")
    rpt = npad // NS
    nbc = jnp.where(c == 0, nb0, nb1)
    base = jnp.where(c == 0, s * nb0, NS * nb0 + s * nb1) * EB
    _zero_rows(rows0, EB, 128)
    @pl.loop(0, rpt // EB)
    def _(k):
        pltpu.sync_copy(rows0, acc.at[pl.ds(s * rpt + k * EB, EB)])
    plsc.subcore_barrier()

    ridx0, ridx1 = ridx
    cidx0, cidx1 = cidx
    ewb0, ewb1 = ewb

    def scale(rows, ew):
        @pl.loop(0, EB // L)
        def _(g):
            w16 = ew[pl.ds(g * L, L)]
            for j in range(L):
                w = w16[j]
                r = g * L + j
                for f in range(8):
                    sl = pl.ds(f * L, L)
                    rows[r, sl] = rows[r, sl] * w

    def start(b, ri, ci, ew, rows, sem):
        eb = base + b * EB
        pltpu.sync_copy(row_hbm.at[pl.ds(eb, EB)], ri)
        pltpu.sync_copy(col_hbm.at[pl.ds(eb, EB)], ci)
        pltpu.sync_copy(ew_hbm.at[pl.ds(eb, EB)], ew)
        pltpu.async_copy(y_hbm.at[ri], rows, sem)

    def wait(ri, rows, sem):
        pltpu.make_async_copy(y_hbm.at[ri], rows, sem).wait()

    start(0, ridx0, cidx0, ewb0, rows0, sem0)
    @pl.loop(0, nbc // 2)
    def _(k):
        b0 = 2 * k
        start(b0 + 1, ridx1, cidx1, ewb1, rows1, sem1)
        wait(ridx0, rows0, sem0)
        scale(rows0, ewb0)
        pltpu.sync_copy(rows0, acc.at[cidx0], add=True)
        @pl.when(b0 + 2 < nbc)
        def _():
            start(b0 + 2, ridx0, cidx0, ewb0, rows0, sem0)
        wait(ridx1, rows1, sem1)
        scale(rows1, ewb1)
        pltpu.sync_copy(rows1, acc.at[cidx1], add=True)
    @pl.when(nbc % 2 == 1)
    def _():
        wait(ridx0, rows0, sem0)
        scale(rows0, ewb0)
        pltpu.sync_copy(rows0, acc.at[cidx0], add=True)

    plsc.subcore_barrier()
    pltpu.sync_copy(acc.at[pl.ds(s * rpt, rpt)],
                    out_hbm.at[pl.ds(c * npad + s * rpt, rpt)])


def _sc_readout(nb0, nb1, h2_hbm, row_hbm, col_hbm, hr_hbm, hc_hbm,
                ridx, cidx, bufr0, bufc0, bufr1, bufc1, semr0, semc0,
                semr1, semc1):
    c = lax.axis_index("c")
    s = lax.axis_index("s")
    nbc = jnp.where(c == 0, nb0, nb1)
    base = jnp.where(c == 0, s * nb0, NS * nb0 + s * nb1) * EB

    @pl.when(c == 0)
    def _():
        pltpu.sync_copy(row_hbm.at[pl.ds(base, nb0 * EB)],
                        ridx.at[pl.ds(0, nb0 * EB)])
        pltpu.sync_copy(col_hbm.at[pl.ds(base, nb0 * EB)],
                        cidx.at[pl.ds(0, nb0 * EB)])
    @pl.when(c == 1)
    def _():
        pltpu.sync_copy(row_hbm.at[pl.ds(base, nb1 * EB)],
                        ridx.at[pl.ds(0, nb1 * EB)])
        pltpu.sync_copy(col_hbm.at[pl.ds(base, nb1 * EB)],
                        cidx.at[pl.ds(0, nb1 * EB)])

    def start(b, br, bc, sr, sc_):
        pltpu.async_copy(h2_hbm.at[ridx.at[pl.ds(b * EB, EB)]], br, sr)
        pltpu.async_copy(h2_hbm.at[cidx.at[pl.ds(b * EB, EB)]], bc, sc_)

    def finish(b, br, bc, sr, sc_):
        pltpu.make_async_copy(h2_hbm.at[ridx.at[pl.ds(0, EB)]], br, sr).wait()
        pltpu.make_async_copy(h2_hbm.at[cidx.at[pl.ds(0, EB)]], bc, sc_).wait()
        pltpu.sync_copy(br, hr_hbm.at[pl.ds(base + b * EB, EB)])
        pltpu.sync_copy(bc, hc_hbm.at[pl.ds(base + b * EB, EB)])

    start(0, bufr0, bufc0, semr0, semc0)
    @pl.loop(0, nbc // 2)
    def _(k):
        b0 = 2 * k
        start(b0 + 1, bufr1, bufc1, semr1, semc1)
        finish(b0, bufr0, bufc0, semr0, semc0)
        @pl.when(b0 + 2 < nbc)
        def _():
            start(b0 + 2, bufr0, bufc0, semr0, semc0)
        finish(b0 + 1, bufr1, bufc1, semr1, semc1)
    @pl.when(nbc % 2 == 1)
    def _():
        finish(nbc - 1, bufr0, bufc0, semr0, semc0)


def _leaky(x):
    return jnp.where(x > 0, x, 0.01 * x)


def _dinv_of(dg0, dg1):
    deg = dg0[...] + dg1[...] + 1.0
    return jnp.where(deg > 0, lax.rsqrt(deg), 0.0)


def _ln(x, g, b):
    mu = jnp.mean(x, axis=-1, keepdims=True)
    var = jnp.mean((x - mu) * (x - mu), axis=-1, keepdims=True)
    return (x - mu) * lax.rsqrt(var + 1e-5) * g + b


def _tc_a(x_ref, dg0, dg1, g1, b1, w0, y0_out):
    xb = x_ref[...]
    ln = _ln(xb, g1[...], b1[...])
    xw = jnp.dot(ln, w0[...], preferred_element_type=jnp.float32)
    dinv = _dinv_of(dg0[...], dg1[...])
    y0_out[...] = dinv[:, None] * xw


def _tc_b(a0, a1, y0, dg0, dg1, b0, w1, h_out, y1_out):
    dinv = _dinv_of(dg0[...], dg1[...])
    hpre = dinv[:, None] * (a0[...] + a1[...] + y0[...]) + b0[...]
    h = _leaky(hpre)
    h_out[...] = h
    xw1 = jnp.dot(h, w1[...], preferred_element_type=jnp.float32)
    y1_out[...] = dinv[:, None] * xw1


def _tc_c(a0, a1, y1, h_ref, dg0, dg1, b1, n1g, n1b, h2_out):
    dinv = _dinv_of(dg0[...], dg1[...])
    g = dinv[:, None] * (a0[...] + a1[...] + y1[...]) + b1[...]
    h2_out[...] = _leaky(_ln(g, n1g[...], n1b[...]) + h_ref[...])


def _tc_d(hr, hc, lg2, lb2, we, be, wf, bf, out):
    ef = jnp.concatenate([hr[...], hc[...]], axis=1)
    ln = _ln(ef, lg2[...], lb2[...])
    eo1 = _leaky(jnp.dot(ln, we[...], preferred_element_type=jnp.float32)
                 + be[...])
    o = jnp.sum(eo1 * wf[...], axis=1)
    out[...] = o[:, None] + bf[0, 0]


def kernel(x, edge_index, edge_attr, ln1_g, ln1_b, W0, b0, W1, b1,
           n1_g, n1_b, ln2_g, ln2_b, We, be, Wf, bf):
    N, D = x.shape
    E = edge_attr.shape[0]
    H = W0.shape[1]
    npad = ((N + NS * EB - 1) // (NS * EB)) * (NS * EB)
    e_pad = ((E + NW * EB - 1) // (NW * EB)) * (NW * EB)
    ept = e_pad // NW
    nb = ept // EB

    f32 = jnp.float32
    rowp = jnp.pad(edge_index[0], (0, e_pad - E))
    colp = jnp.pad(edge_index[1], (0, e_pad - E))
    ewp = jnp.pad(edge_attr, (0, e_pad - E))
    xp = jnp.pad(x, ((0, npad - N), (0, 0)))

    mesh = plsc.VectorSubcoreMesh(core_axis_name="c", subcore_axis_name="s",
                                  num_cores=NC, num_subcores=NS)

    deg_parts = pl.kernel(
        functools.partial(_sc_deg, npad, ept, nb),
        out_type=jax.ShapeDtypeStruct((NC * npad,), f32),
        mesh=mesh,
        scratch_types=[
            pltpu.VMEM_SHARED((npad,), f32),
            pltpu.VMEM((EB,), jnp.int32),
            pltpu.VMEM((EB,), f32),
            pltpu.VMEM((EB,), f32),
            pltpu.SemaphoreType.DMA,
        ],
    )(colp, ewp)
    dg0 = deg_parts[:npad]
    dg1 = deg_parts[npad:]

    nblocks = npad // NBLK
    bspec_n = pl.BlockSpec((NBLK, H), lambda i: (i, 0))
    bspec_16 = pl.BlockSpec((NBLK,), lambda i: (i,))
    bspec_st = pl.BlockSpec((NBLK, 16), lambda i: (i, 0))
    bspec_p = pl.BlockSpec((1, H), lambda i: (0, 0))
    bspec_w = pl.BlockSpec((D, H), lambda i: (0, 0))

    y0 = pl.pallas_call(
        _tc_a,
        grid=(nblocks,),
        in_specs=[bspec_n, bspec_16, bspec_16, bspec_p, bspec_p, bspec_w],
        out_specs=bspec_n,
        out_shape=jax.ShapeDtypeStruct((npad, H), f32),
    )(xp, dg0, dg1, ln1_g.reshape(1, -1), ln1_b.reshape(1, -1), W0)

    nbt = e_pad // (NS * EB)
    mp0_share = (2 * nbt) // 5
    ro0_share = (8 * nbt) // 25

    def mp(y):
        parts = pl.kernel(
            functools.partial(_sc_mp, npad, mp0_share, nbt - mp0_share),
            out_type=jax.ShapeDtypeStruct((NC * npad, H), f32),
            mesh=mesh,
            scratch_types=[
                pltpu.VMEM_SHARED((npad, H), f32),
                (pltpu.VMEM((EB,), jnp.int32), pltpu.VMEM((EB,), jnp.int32)),
                (pltpu.VMEM((EB,), jnp.int32), pltpu.VMEM((EB,), jnp.int32)),
                (pltpu.VMEM((EB,), f32), pltpu.VMEM((EB,), f32)),
                pltpu.VMEM((EB, H), f32),
                pltpu.VMEM((EB, H), f32),
                pltpu.SemaphoreType.DMA,
                pltpu.SemaphoreType.DMA,
            ],
        )(y, rowp, colp, ewp)
        return parts[:npad], parts[npad:]

    a0, a1 = mp(y0)
    h, y1 = pl.pallas_call(
        _tc_b,
        grid=(nblocks,),
        in_specs=[bspec_n, bspec_n, bspec_n, bspec_16, bspec_16, bspec_p,
                  bspec_w],
        out_specs=[bspec_n, bspec_n],
        out_shape=[jax.ShapeDtypeStruct((npad, H), f32),
                   jax.ShapeDtypeStruct((npad, H), f32)],
    )(a0, a1, y0, dg0, dg1, b0.reshape(1, -1), W1)

    c0, c1 = mp(y1)
    h2 = pl.pallas_call(
        _tc_c,
        grid=(nblocks,),
        in_specs=[bspec_n, bspec_n, bspec_n, bspec_n, bspec_16, bspec_16,
                  bspec_p, bspec_p, bspec_p],
        out_specs=bspec_n,
        out_shape=jax.ShapeDtypeStruct((npad, H), f32),
    )(c0, c1, y1, h, dg0, dg1, b1.reshape(1, -1), n1_g.reshape(1, -1),
      n1_b.reshape(1, -1))

    ro_max = max(ro0_share, nbt - ro0_share) * EB
    hr, hc = pl.kernel(
        functools.partial(_sc_readout, ro0_share, nbt - ro0_share),
        out_type=[jax.ShapeDtypeStruct((e_pad, H), f32),
                  jax.ShapeDtypeStruct((e_pad, H), f32)],
        mesh=mesh,
        scratch_types=[
            pltpu.VMEM((ro_max,), jnp.int32),
            pltpu.VMEM((ro_max,), jnp.int32),
            pltpu.VMEM((EB, H), f32),
            pltpu.VMEM((EB, H), f32),
            pltpu.VMEM((EB, H), f32),
            pltpu.VMEM((EB, H), f32),
            pltpu.SemaphoreType.DMA,
            pltpu.SemaphoreType.DMA,
            pltpu.SemaphoreType.DMA,
            pltpu.SemaphoreType.DMA,
        ],
    )(h2, rowp, colp)

    eblk = 2048
    egrid = e_pad // eblk
    espec = pl.BlockSpec((eblk, H), lambda i: (i, 0))
    cspec = pl.BlockSpec((1, 2 * H), lambda i: (0, 0))
    eo = pl.pallas_call(
        _tc_d,
        grid=(egrid,),
        in_specs=[espec, espec, cspec, cspec,
                  pl.BlockSpec((2 * H, H), lambda i: (0, 0)),
                  pl.BlockSpec((1, H), lambda i: (0, 0)),
                  pl.BlockSpec((1, H), lambda i: (0, 0)),
                  pl.BlockSpec((1, 1), lambda i: (0, 0))],
        out_specs=pl.BlockSpec((eblk, 1), lambda i: (i, 0)),
        out_shape=jax.ShapeDtypeStruct((e_pad, 1), f32),
    )(hr, hc, ln2_g.reshape(1, -1), ln2_b.reshape(1, -1), We,
      be.reshape(1, -1), Wf.reshape(1, -1), bf.reshape(1, 1))

    return eo[:E]
